# Initial kernel scaffold; baseline (speedup 1.0000x reference)
import jax
import jax.numpy as jnp
from jax.experimental import pallas as pl


def kernel(train_inputs, train_types, node_neigh, node_embeddings, node_type_embeddings, trans_weights, trans_weights_s1, trans_weights_s2):
    raise NotImplementedError("write your pallas kernel here")


# speedup vs baseline: 4.2481x; 4.2481x over previous
import functools

import jax
import jax.numpy as jnp
from jax import lax
from jax.experimental import pallas as pl
from jax.experimental.pallas import tpu as pltpu
from jax.experimental.pallas import tpu_sc as plsc

_NN = 50000
_E = 128
_U = 32
_L = 4
_B = 4096
_NS = 10

_NW = 32
_RPW = _B // _NW
_BB = 16
_NBLK = _RPW // _BB
_GPB = _BB * _L * _NS
_GC = 128
_NGC = _GPB // _GC
_IDXN = _RPW * _L * _NS


def _sc_body(neigh_hbm, ti_hbm, nte_hbm, nemb_hbm,
             s_hbm, base_hbm,
             neigh_v, buf_v, sbuf_v, ti_v, bbuf_v, sem):
    c = lax.axis_index("c")
    s = lax.axis_index("s")
    wid = s * 2 + c
    row0 = wid * _RPW

    pltpu.sync_copy(neigh_hbm.at[pl.ds(row0 * _L * _NS, _IDXN)], neigh_v)
    pltpu.sync_copy(ti_hbm.at[pl.ds(row0, _RPW)], ti_v)

    for blk in range(_NBLK):
        gbase = blk * _GPB
        cps = [
            pltpu.async_copy(
                nte_hbm.at[neigh_v.at[pl.ds(gbase + k * _GC, _GC)]],
                buf_v.at[pl.ds(k * _GC, _GC)],
                sem,
            )
            for k in range(_NGC)
        ]
        for cp in cps:
            cp.wait()

        def acc_body(d, carry):
            r0 = d * _NS
            l = d % _L
            col = l * _U
            a0 = buf_v[r0, pl.ds(col, 16)]
            a1 = buf_v[r0, pl.ds(col + 16, 16)]
            for r in range(1, _NS):
                a0 = a0 + buf_v[r0 + r, pl.ds(col, 16)]
                a1 = a1 + buf_v[r0 + r, pl.ds(col + 16, 16)]
            b_loc = d // _L
            sbuf_v[blk * _BB + b_loc, pl.ds(col, 16)] = a0
            sbuf_v[blk * _BB + b_loc, pl.ds(col + 16, 16)] = a1
            return carry

        lax.fori_loop(0, _BB * _L, acc_body, 0)

    pltpu.async_copy(nemb_hbm.at[ti_v], bbuf_v, sem).wait()
    pltpu.sync_copy(sbuf_v, s_hbm.at[pl.ds(row0, _RPW)])
    pltpu.sync_copy(bbuf_v, base_hbm.at[pl.ds(row0, _RPW)])


def _sc_stage(neigh_flat, ti, nte_flat, nemb):
    f = pl.kernel(
        _sc_body,
        out_type=(
            jax.ShapeDtypeStruct((_B, _E), jnp.float32),
            jax.ShapeDtypeStruct((_B, _E), jnp.float32),
        ),
        mesh=plsc.VectorSubcoreMesh(core_axis_name="c", subcore_axis_name="s"),
        scratch_types=[
            pltpu.VMEM((_IDXN,), jnp.int32),
            pltpu.VMEM((_GPB, _L * _U), jnp.float32),
            pltpu.VMEM((_RPW, _E), jnp.float32),
            pltpu.VMEM((_RPW,), jnp.int32),
            pltpu.VMEM((_RPW, _E), jnp.float32),
            pltpu.SemaphoreType.DMA,
        ],
    )
    return f(neigh_flat, ti, nte_flat, nemb)


_TB = 256
_NTB = _B // _TB


def _tc_body(ti_col_ref, ti_row_ref, tt_ref, s_ref, base_ref,
             w1_ref, w2_ref, w_ref, o_ref):
    tcol = ti_col_ref[...]

    def jb(j, acc):
        trow = ti_row_ref[:, pl.ds(j * _TB, _TB)]
        eq = (tcol == trow).astype(jnp.float32)
        sj = s_ref[pl.ds(j * _TB, _TB), :]
        return acc + jnp.dot(eq, sj, preferred_element_type=jnp.float32)

    acc = lax.fori_loop(0, _NTB, jb, jnp.zeros((_TB, _E), jnp.float32))

    tt = tt_ref[...]
    nte_l = [acc[:, l * _U:(l + 1) * _U] for l in range(_L)]
    s_l = [jnp.zeros((_TB, 1), jnp.float32) for _ in range(_L)]
    for t in range(_L):
        selt = tt == t
        w1 = w1_ref[t]
        w2row = w2_ref[pl.ds(t, 1), :]
        for l in range(_L):
            h = jnp.dot(nte_l[l], w1, preferred_element_type=jnp.float32)
            sc = jnp.sum(jnp.tanh(h) * w2row, axis=1, keepdims=True)
            s_l[l] = jnp.where(selt, sc, s_l[l])
    m = jnp.maximum(jnp.maximum(s_l[0], s_l[1]), jnp.maximum(s_l[2], s_l[3]))
    e = [jnp.exp(x - m) for x in s_l]
    z = e[0] + e[1] + e[2] + e[3]
    att = [x / z for x in e]
    nta = (att[0] * nte_l[0] + att[1] * nte_l[1]
           + att[2] * nte_l[2] + att[3] * nte_l[3])
    proj = jnp.zeros((_TB, _E), jnp.float32)
    for t in range(_L):
        zt = jnp.dot(nta, w_ref[t], preferred_element_type=jnp.float32)
        proj = jnp.where(tt == t, zt, proj)
    pre = base_ref[...] + proj
    nrm = jnp.sqrt(jnp.sum(pre * pre, axis=1, keepdims=True))
    o_ref[...] = pre / jnp.maximum(nrm, 1e-12)


def _tc_stage(ti_col, ti_row, tt_col, s, base, w1, w2p, w):
    return pl.pallas_call(
        _tc_body,
        grid=(_NTB,),
        in_specs=[
            pl.BlockSpec((_TB, 1), lambda i: (i, 0)),
            pl.BlockSpec((1, _B), lambda i: (0, 0)),
            pl.BlockSpec((_TB, 1), lambda i: (i, 0)),
            pl.BlockSpec((_B, _E), lambda i: (0, 0)),
            pl.BlockSpec((_TB, _E), lambda i: (i, 0)),
            pl.BlockSpec((_L, _U, _U), lambda i: (0, 0, 0)),
            pl.BlockSpec((8, _U), lambda i: (0, 0)),
            pl.BlockSpec((_L, _U, _E), lambda i: (0, 0, 0)),
        ],
        out_specs=pl.BlockSpec((_TB, _E), lambda i: (i, 0)),
        out_shape=jax.ShapeDtypeStruct((_B, _E), jnp.float32),
        compiler_params=pltpu.CompilerParams(
            dimension_semantics=("arbitrary",),
        ),
    )(ti_col, ti_row, tt_col, s, base, w1, w2p, w)


def kernel(train_inputs, train_types, node_neigh, node_embeddings,
           node_type_embeddings, trans_weights, trans_weights_s1,
           trans_weights_s2):
    ti = train_inputs.astype(jnp.int32)
    tt = train_types.astype(jnp.int32)
    neigh_flat = node_neigh.astype(jnp.int32).reshape(-1)
    nte_flat = node_type_embeddings.reshape(_NN, _L * _U)
    s, base = _sc_stage(neigh_flat, ti, nte_flat, node_embeddings)
    w2p = jnp.zeros((8, _U), jnp.float32).at[:_L].set(trans_weights_s2[:, :, 0])
    return _tc_stage(ti.reshape(_B, 1), ti.reshape(1, _B), tt.reshape(_B, 1),
                     s, base, trans_weights_s1, w2p, trans_weights)


# speedup vs baseline: 4.5995x; 1.0827x over previous
import functools

import jax
import jax.numpy as jnp
from jax import lax
from jax.experimental import pallas as pl
from jax.experimental.pallas import tpu as pltpu
from jax.experimental.pallas import tpu_sc as plsc

_NN = 50000
_E = 128
_U = 32
_L = 4
_B = 4096
_NS = 10

_NW = 32
_RPW = _B // _NW
_BB = 16
_NBLK = _RPW // _BB
_GPB = _BB * _L * _NS
_GC = 128
_NGC = _GPB // _GC
_IDXN = _RPW * _L * _NS


def _sc_body(neigh_hbm, ti_hbm, nte_hbm, nemb_hbm,
             s_hbm, base_hbm,
             neigh_v, buf_v, sbuf_v, ti_v, bbuf_v, sem):
    c = lax.axis_index("c")
    s = lax.axis_index("s")
    wid = s * 2 + c
    row0 = wid * _RPW

    pltpu.sync_copy(neigh_hbm.at[pl.ds(row0 * _L * _NS, _IDXN)], neigh_v)
    pltpu.sync_copy(ti_hbm.at[pl.ds(row0, _RPW)], ti_v)

    for blk in range(_NBLK):
        gbase = blk * _GPB
        cps = [
            pltpu.async_copy(
                nte_hbm.at[neigh_v.at[pl.ds(gbase + k * _GC, _GC)]],
                buf_v.at[pl.ds(k * _GC, _GC)],
                sem,
            )
            for k in range(_NGC)
        ]
        for cp in cps:
            cp.wait()

        def acc_body(d, carry):
            r0 = d * _NS
            l = d % _L
            col = l * _U
            a0 = buf_v[r0, pl.ds(col, 16)]
            a1 = buf_v[r0, pl.ds(col + 16, 16)]
            for r in range(1, _NS):
                a0 = a0 + buf_v[r0 + r, pl.ds(col, 16)]
                a1 = a1 + buf_v[r0 + r, pl.ds(col + 16, 16)]
            b_loc = d // _L
            sbuf_v[blk * _BB + b_loc, pl.ds(col, 16)] = a0
            sbuf_v[blk * _BB + b_loc, pl.ds(col + 16, 16)] = a1
            return carry

        lax.fori_loop(0, _BB * _L, acc_body, 0)

    pltpu.async_copy(nemb_hbm.at[ti_v], bbuf_v, sem).wait()
    pltpu.sync_copy(sbuf_v, s_hbm.at[pl.ds(row0, _RPW)])
    pltpu.sync_copy(bbuf_v, base_hbm.at[pl.ds(row0, _RPW)])


def _sc_stage(neigh_flat, ti, nte_flat, nemb):
    f = pl.kernel(
        _sc_body,
        out_type=(
            jax.ShapeDtypeStruct((_B, _E), jnp.float32),
            jax.ShapeDtypeStruct((_B, _E), jnp.float32),
        ),
        mesh=plsc.VectorSubcoreMesh(core_axis_name="c", subcore_axis_name="s"),
        scratch_types=[
            pltpu.VMEM((_IDXN,), jnp.int32),
            pltpu.VMEM((_GPB, _L * _U), jnp.float32),
            pltpu.VMEM((_RPW, _E), jnp.float32),
            pltpu.VMEM((_RPW,), jnp.int32),
            pltpu.VMEM((_RPW, _E), jnp.float32),
            pltpu.SemaphoreType.DMA,
        ],
    )
    return f(neigh_flat, ti, nte_flat, nemb)


_TB = 256
_NTB = _B // _TB


def _tc_body(ti_col_ref, ti_row_ref, tt_ref, shi_ref, slo_ref, base_ref,
             w1c_ref, w2c_ref, wc_ref, o_ref):
    tcol = ti_col_ref[...]
    trow = ti_row_ref[...]
    eq = (tcol == trow).astype(jnp.bfloat16)
    acc = (jnp.dot(eq, shi_ref[...], preferred_element_type=jnp.float32)
           + jnp.dot(eq, slo_ref[...], preferred_element_type=jnp.float32))

    tt = tt_ref[...]
    w2c = w2c_ref[...]
    nte_l = [acc[:, l * _U:(l + 1) * _U] for l in range(_L)]
    att_s = []
    for l in range(_L):
        h = jnp.dot(nte_l[l], w1c_ref[...],
                    preferred_element_type=jnp.float32)
        g = jnp.tanh(h) * w2c
        sc = jnp.zeros((_TB, 1), jnp.float32)
        for t in range(_L):
            cs = jnp.sum(g[:, t * _U:(t + 1) * _U], axis=1, keepdims=True)
            sc = jnp.where(tt == t, cs, sc)
        att_s.append(sc)
    m = jnp.maximum(jnp.maximum(att_s[0], att_s[1]),
                    jnp.maximum(att_s[2], att_s[3]))
    e = [jnp.exp(x - m) for x in att_s]
    z = e[0] + e[1] + e[2] + e[3]
    nta = ((e[0] / z) * nte_l[0] + (e[1] / z) * nte_l[1]
           + (e[2] / z) * nte_l[2] + (e[3] / z) * nte_l[3])
    z_all = jnp.dot(nta, wc_ref[...],
                    preferred_element_type=jnp.float32)
    proj = jnp.zeros((_TB, _E), jnp.float32)
    for t in range(_L):
        proj = jnp.where(tt == t, z_all[:, t * _E:(t + 1) * _E], proj)
    pre = base_ref[...] + proj
    nrm = jnp.sqrt(jnp.sum(pre * pre, axis=1, keepdims=True))
    o_ref[...] = pre / jnp.maximum(nrm, 1e-12)


def _tc_stage(ti_col, ti_row, tt_col, s_hi, s_lo, base, w1c, w2c, wc):
    return pl.pallas_call(
        _tc_body,
        grid=(_NTB,),
        in_specs=[
            pl.BlockSpec((_TB, 1), lambda i: (i, 0)),
            pl.BlockSpec((1, _B), lambda i: (0, 0)),
            pl.BlockSpec((_TB, 1), lambda i: (i, 0)),
            pl.BlockSpec((_B, _E), lambda i: (0, 0)),
            pl.BlockSpec((_B, _E), lambda i: (0, 0)),
            pl.BlockSpec((_TB, _E), lambda i: (i, 0)),
            pl.BlockSpec((_U, _L * _U), lambda i: (0, 0)),
            pl.BlockSpec((1, _L * _U), lambda i: (0, 0)),
            pl.BlockSpec((_U, _L * _E), lambda i: (0, 0)),
        ],
        out_specs=pl.BlockSpec((_TB, _E), lambda i: (i, 0)),
        out_shape=jax.ShapeDtypeStruct((_B, _E), jnp.float32),
        compiler_params=pltpu.CompilerParams(
            dimension_semantics=("arbitrary",),
        ),
    )(ti_col, ti_row, tt_col, s_hi, s_lo, base, w1c, w2c, wc)


def kernel(train_inputs, train_types, node_neigh, node_embeddings,
           node_type_embeddings, trans_weights, trans_weights_s1,
           trans_weights_s2):
    ti = train_inputs.astype(jnp.int32)
    tt = train_types.astype(jnp.int32)
    neigh_flat = node_neigh.astype(jnp.int32).reshape(-1)
    nte_flat = node_type_embeddings.reshape(_NN, _L * _U)
    s, base = _sc_stage(neigh_flat, ti, nte_flat, node_embeddings)
    s_hi = s.astype(jnp.bfloat16)
    s_lo = (s - s_hi.astype(jnp.float32)).astype(jnp.bfloat16)
    w1c = jnp.concatenate([trans_weights_s1[t] for t in range(_L)], axis=1)
    w2c = trans_weights_s2[:, :, 0].reshape(1, _L * _U)
    wc = jnp.concatenate([trans_weights[t] for t in range(_L)], axis=1)
    return _tc_stage(ti.reshape(_B, 1), ti.reshape(1, _B), tt.reshape(_B, 1),
                     s_hi, s_lo, base, w1c, w2c, wc)


# speedup vs baseline: 4.7359x; 1.0297x over previous
import functools

import jax
import jax.numpy as jnp
from jax import lax
from jax.experimental import pallas as pl
from jax.experimental.pallas import tpu as pltpu
from jax.experimental.pallas import tpu_sc as plsc

_NN = 50000
_E = 128
_U = 32
_L = 4
_B = 4096
_NS = 10

_NW = 32
_RPW = _B // _NW
_BB = 8
_NBLK = _RPW // _BB
_GPB = _BB * _L * _NS
_GCS = (128, 128, 64)
_IDXN = _RPW * _L * _NS


def _sc_body(neigh_hbm, ti_hbm, nte_hbm, nemb_hbm,
             s_hbm, base_hbm,
             neigh_v, buf0_v, buf1_v, sbuf_v, ti_v, bbuf_v,
             sem0, sem1, bsem):
    c = lax.axis_index("c")
    s = lax.axis_index("s")
    wid = s * 2 + c
    row0 = wid * _RPW

    pltpu.sync_copy(neigh_hbm.at[pl.ds(row0 * _L * _NS, _IDXN)], neigh_v)
    pltpu.sync_copy(ti_hbm.at[pl.ds(row0, _RPW)], ti_v)
    base_cp = pltpu.async_copy(nemb_hbm.at[ti_v], bbuf_v, bsem)

    bufs = (buf0_v, buf1_v)
    sems = (sem0, sem1)

    def fire(blk):
        gbase = blk * _GPB
        buf = bufs[blk % 2]
        sem = sems[blk % 2]
        cps = []
        off = 0
        for gc in _GCS:
            cps.append(pltpu.async_copy(
                nte_hbm.at[neigh_v.at[pl.ds(gbase + off, gc)]],
                buf.at[pl.ds(off, gc)],
                sem,
            ))
            off += gc
        return cps

    pend = fire(0)
    for blk in range(_NBLK):
        nxt = fire(blk + 1) if blk + 1 < _NBLK else []
        for cp in pend:
            cp.wait()
        pend = nxt
        buf = bufs[blk % 2]

        def acc_body(d, carry):
            r0 = d * _NS
            l = d % _L
            col = l * _U
            a0 = buf[r0, pl.ds(col, 16)]
            a1 = buf[r0, pl.ds(col + 16, 16)]
            for r in range(1, _NS):
                a0 = a0 + buf[r0 + r, pl.ds(col, 16)]
                a1 = a1 + buf[r0 + r, pl.ds(col + 16, 16)]
            b_loc = d // _L
            sbuf_v[blk * _BB + b_loc, pl.ds(col, 16)] = a0
            sbuf_v[blk * _BB + b_loc, pl.ds(col + 16, 16)] = a1
            return carry

        lax.fori_loop(0, _BB * _L, acc_body, 0)

    base_cp.wait()
    pltpu.sync_copy(sbuf_v, s_hbm.at[pl.ds(row0, _RPW)])
    pltpu.sync_copy(bbuf_v, base_hbm.at[pl.ds(row0, _RPW)])


def _sc_stage(neigh_flat, ti, nte_flat, nemb):
    f = pl.kernel(
        _sc_body,
        out_type=(
            jax.ShapeDtypeStruct((_B, _E), jnp.float32),
            jax.ShapeDtypeStruct((_B, _E), jnp.float32),
        ),
        mesh=plsc.VectorSubcoreMesh(core_axis_name="c", subcore_axis_name="s"),
        scratch_types=[
            pltpu.VMEM((_IDXN,), jnp.int32),
            pltpu.VMEM((_GPB, _L * _U), jnp.float32),
            pltpu.VMEM((_GPB, _L * _U), jnp.float32),
            pltpu.VMEM((_RPW, _E), jnp.float32),
            pltpu.VMEM((_RPW,), jnp.int32),
            pltpu.VMEM((_RPW, _E), jnp.float32),
            pltpu.SemaphoreType.DMA,
            pltpu.SemaphoreType.DMA,
            pltpu.SemaphoreType.DMA,
        ],
    )
    return f(neigh_flat, ti, nte_flat, nemb)


_TB = 256
_NTB = _B // _TB


def _tc_body(ti_col_ref, ti_row_ref, tt_ref, shi_ref, slo_ref, base_ref,
             w1c_ref, w2c_ref, wc_ref, o_ref):
    tcol = ti_col_ref[...]
    trow = ti_row_ref[...]
    eq = (tcol == trow).astype(jnp.bfloat16)
    acc = (jnp.dot(eq, shi_ref[...], preferred_element_type=jnp.float32)
           + jnp.dot(eq, slo_ref[...], preferred_element_type=jnp.float32))

    tt = tt_ref[...]
    w2c = w2c_ref[...]
    nte_l = [acc[:, l * _U:(l + 1) * _U] for l in range(_L)]
    att_s = []
    for l in range(_L):
        h = jnp.dot(nte_l[l], w1c_ref[...],
                    preferred_element_type=jnp.float32)
        g = jnp.tanh(h) * w2c
        sc = jnp.zeros((_TB, 1), jnp.float32)
        for t in range(_L):
            cs = jnp.sum(g[:, t * _U:(t + 1) * _U], axis=1, keepdims=True)
            sc = jnp.where(tt == t, cs, sc)
        att_s.append(sc)
    m = jnp.maximum(jnp.maximum(att_s[0], att_s[1]),
                    jnp.maximum(att_s[2], att_s[3]))
    e = [jnp.exp(x - m) for x in att_s]
    z = e[0] + e[1] + e[2] + e[3]
    nta = ((e[0] / z) * nte_l[0] + (e[1] / z) * nte_l[1]
           + (e[2] / z) * nte_l[2] + (e[3] / z) * nte_l[3])
    z_all = jnp.dot(nta, wc_ref[...],
                    preferred_element_type=jnp.float32)
    proj = jnp.zeros((_TB, _E), jnp.float32)
    for t in range(_L):
        proj = jnp.where(tt == t, z_all[:, t * _E:(t + 1) * _E], proj)
    pre = base_ref[...] + proj
    nrm = jnp.sqrt(jnp.sum(pre * pre, axis=1, keepdims=True))
    o_ref[...] = pre / jnp.maximum(nrm, 1e-12)


def _tc_stage(ti_col, ti_row, tt_col, s_hi, s_lo, base, w1c, w2c, wc):
    return pl.pallas_call(
        _tc_body,
        grid=(_NTB,),
        in_specs=[
            pl.BlockSpec((_TB, 1), lambda i: (i, 0)),
            pl.BlockSpec((1, _B), lambda i: (0, 0)),
            pl.BlockSpec((_TB, 1), lambda i: (i, 0)),
            pl.BlockSpec((_B, _E), lambda i: (0, 0)),
            pl.BlockSpec((_B, _E), lambda i: (0, 0)),
            pl.BlockSpec((_TB, _E), lambda i: (i, 0)),
            pl.BlockSpec((_U, _L * _U), lambda i: (0, 0)),
            pl.BlockSpec((1, _L * _U), lambda i: (0, 0)),
            pl.BlockSpec((_U, _L * _E), lambda i: (0, 0)),
        ],
        out_specs=pl.BlockSpec((_TB, _E), lambda i: (i, 0)),
        out_shape=jax.ShapeDtypeStruct((_B, _E), jnp.float32),
        compiler_params=pltpu.CompilerParams(
            dimension_semantics=("arbitrary",),
        ),
    )(ti_col, ti_row, tt_col, s_hi, s_lo, base, w1c, w2c, wc)


def kernel(train_inputs, train_types, node_neigh, node_embeddings,
           node_type_embeddings, trans_weights, trans_weights_s1,
           trans_weights_s2):
    ti = train_inputs.astype(jnp.int32)
    tt = train_types.astype(jnp.int32)
    neigh_flat = node_neigh.astype(jnp.int32).reshape(-1)
    nte_flat = node_type_embeddings.reshape(_NN, _L * _U)
    s, base = _sc_stage(neigh_flat, ti, nte_flat, node_embeddings)
    s_hi = s.astype(jnp.bfloat16)
    s_lo = (s - s_hi.astype(jnp.float32)).astype(jnp.bfloat16)
    w1c = jnp.concatenate([trans_weights_s1[t] for t in range(_L)], axis=1)
    w2c = trans_weights_s2[:, :, 0].reshape(1, _L * _U)
    wc = jnp.concatenate([trans_weights[t] for t in range(_L)], axis=1)
    return _tc_stage(ti.reshape(_B, 1), ti.reshape(1, _B), tt.reshape(_B, 1),
                     s_hi, s_lo, base, w1c, w2c, wc)
